# Initial kernel scaffold; baseline (speedup 1.0000x reference)
import jax
import jax.numpy as jnp
from jax.experimental import pallas as pl


def kernel(xyz, xyz_batch_cnt, new_xyz, new_xyz_batch_cnt, features):
    raise NotImplementedError("write your pallas kernel here")


# speedup vs baseline: 1.4281x; 1.4281x over previous
import functools

import jax
import jax.numpy as jnp
from jax import lax
from jax.experimental import pallas as pl
from jax.experimental.pallas import tpu as pltpu
from jax.experimental.pallas import tpu_sc as plsc

R0, R1 = 0.08, 0.16
K0, K1 = 16, 32
N = 16384
M = 2048
C = 64
NC, NS = 2, 16
NW = NC * NS
QPW = M // NW
L = 16


def _sc_group_pool(xs, ys, zs, qx, qy, qz, qstart, qend, features):
    mesh = plsc.VectorSubcoreMesh(core_axis_name="c", subcore_axis_name="s")

    @functools.partial(
        pl.kernel,
        out_type=jax.ShapeDtypeStruct((M * 2 * C,), jnp.float32),
        mesh=mesh,
        compiler_params=pltpu.CompilerParams(
            needs_layout_passes=False, use_tc_tiling_on_sc=False),
        scratch_types=[
            pltpu.VMEM((N,), jnp.float32),
            pltpu.VMEM((N,), jnp.float32),
            pltpu.VMEM((N,), jnp.float32),
            pltpu.VMEM((QPW + L,), jnp.float32),
            pltpu.VMEM((QPW + L,), jnp.float32),
            pltpu.VMEM((QPW + L,), jnp.float32),
            pltpu.VMEM((QPW + L,), jnp.int32),
            pltpu.VMEM((QPW + L,), jnp.int32),
            pltpu.VMEM((K0,), jnp.int32),
            pltpu.VMEM((K1,), jnp.int32),
            pltpu.VMEM((K0, C), jnp.float32),
            pltpu.VMEM((K1, C), jnp.float32),
            pltpu.VMEM((QPW * 2 * C,), jnp.float32),
            pltpu.SemaphoreType.DMA,
            pltpu.SemaphoreType.DMA,
        ],
    )
    def grouper(xs_h, ys_h, zs_h, qx_h, qy_h, qz_h, qs_h, qe_h, feat_h,
                out_h, cx, cy, cz, qxv, qyv, qzv, qsv, qev,
                idx0, idx1, rows0, rows1, outbuf, sem0, sem1):
        wid = lax.axis_index("s") * NC + lax.axis_index("c")
        base = wid * QPW

        pltpu.sync_copy(xs_h, cx)
        pltpu.sync_copy(ys_h, cy)
        pltpu.sync_copy(zs_h, cz)
        pltpu.sync_copy(qx_h.at[pl.ds(base, QPW)], qxv.at[pl.ds(0, QPW)])
        pltpu.sync_copy(qy_h.at[pl.ds(base, QPW)], qyv.at[pl.ds(0, QPW)])
        pltpu.sync_copy(qz_h.at[pl.ds(base, QPW)], qzv.at[pl.ds(0, QPW)])
        pltpu.sync_copy(qs_h.at[pl.ds(base, QPW)], qsv.at[pl.ds(0, QPW)])
        pltpu.sync_copy(qe_h.at[pl.ds(base, QPW)], qev.at[pl.ds(0, QPW)])

        iota = lax.iota(jnp.int32, L)
        zeros_i = jnp.zeros((L,), jnp.int32)
        r0sq = jnp.float32(R0 * R0)
        r1sq = jnp.float32(R1 * R1)

        def per_query(m, _):
            qxs = jnp.full((L,), qxv[pl.ds(m, L)][0], jnp.float32)
            qys = jnp.full((L,), qyv[pl.ds(m, L)][0], jnp.float32)
            qzs = jnp.full((L,), qzv[pl.ds(m, L)][0], jnp.float32)
            qs = qsv[pl.ds(m, L)][0]
            qe = qev[pl.ds(m, L)][0]
            qs_v = jnp.full((L,), qs, jnp.int32)
            qe_v = jnp.full((L,), qe, jnp.int32)
            v0 = qs // L
            v1 = (qe + (L - 1)) // L

            def scan_step(v, carry):
                giv, c0v, c1v = carry
                bi = v * L
                xv = cx[pl.ds(bi, L)]
                yv = cy[pl.ds(bi, L)]
                zv = cz[pl.ds(bi, L)]
                dx = xv - qxs
                dy = yv - qys
                dz = zv - qzs
                d2 = dx * dx + dy * dy + dz * dz
                valid = (giv >= qs_v) & (giv < qe_v)
                hit1 = (d2 < r1sq) & valid
                hit0 = (d2 < r0sq) & valid
                inc1 = jnp.cumsum(hit1.astype(jnp.int32))
                inc0 = jnp.cumsum(hit0.astype(jnp.int32))
                p1 = c1v + inc1 - 1
                p0 = c0v + inc0 - 1
                plsc.store_scatter(idx1, [p1], giv, mask=hit1 & (p1 < K1))
                plsc.store_scatter(idx0, [p0], giv, mask=hit0 & (p0 < K0))
                pc1 = plsc.all_reduce_population_count(hit1)
                pc0 = plsc.all_reduce_population_count(hit0)
                return giv + L, c0v + pc0, c1v + pc1

            giv0 = v0 * L + iota
            _giv, c0v, c1v = lax.fori_loop(
                v0, v1, scan_step, (giv0, zeros_i, zeros_i))

            first0 = jnp.full((L,), idx0[pl.ds(0, L)][0], jnp.int32)
            first0 = jnp.where(c0v > 0, first0, 0)
            cl0 = jnp.minimum(c0v, K0)
            cur0 = idx0[pl.ds(0, L)]
            idx0[pl.ds(0, L)] = jnp.where(iota >= cl0, first0, cur0)

            first1 = jnp.full((L,), idx1[pl.ds(0, L)][0], jnp.int32)
            first1 = jnp.where(c1v > 0, first1, 0)
            cl1 = jnp.minimum(c1v, K1)
            for h in range(K1 // L):
                cur1 = idx1[pl.ds(h * L, L)]
                pos = iota + (h * L)
                idx1[pl.ds(h * L, L)] = jnp.where(pos >= cl1, first1, cur1)

            cp0 = pltpu.async_copy(feat_h.at[idx0], rows0, sem0)
            cp1 = pltpu.async_copy(feat_h.at[idx1], rows1, sem1)
            cp0.wait()
            cp1.wait()

            nz0 = c0v > 0
            nz1 = c1v > 0
            obase = m * (2 * C)
            for c in range(C // L):
                acc = rows0[0, pl.ds(c * L, L)]
                for k in range(1, K0):
                    acc = jnp.maximum(acc, rows0[k, pl.ds(c * L, L)])
                outbuf[pl.ds(obase + c * L, L)] = jnp.where(nz0, acc, 0.0)
            for c in range(C // L):
                acc = rows1[0, pl.ds(c * L, L)]
                for k in range(1, K1):
                    acc = jnp.maximum(acc, rows1[k, pl.ds(c * L, L)])
                outbuf[pl.ds(C + obase + c * L, L)] = jnp.where(nz1, acc, 0.0)
            return _

        lax.fori_loop(0, QPW, per_query, 0)
        pltpu.sync_copy(outbuf, out_h.at[pl.ds(base * 2 * C, QPW * 2 * C)])

    return grouper(xs, ys, zs, qx, qy, qz, qstart, qend, features)


def kernel(xyz, xyz_batch_cnt, new_xyz, new_xyz_batch_cnt, features):
    xs = xyz[:, 0]
    ys = xyz[:, 1]
    zs = xyz[:, 2]
    qx = new_xyz[:, 0]
    qy = new_xyz[:, 1]
    qz = new_xyz[:, 2]
    ends = jnp.cumsum(xyz_batch_cnt.astype(jnp.int32))
    starts = ends - xyz_batch_cnt.astype(jnp.int32)
    bid_q = jnp.repeat(
        jnp.arange(new_xyz_batch_cnt.shape[0], dtype=jnp.int32),
        new_xyz_batch_cnt, total_repeat_length=M)
    qstart = starts[bid_q]
    qend = ends[bid_q]
    out = _sc_group_pool(xs, ys, zs, qx, qy, qz, qstart, qend, features)
    return new_xyz, out.reshape(M, 2 * C)
